# baseline (speedup 1.0000x reference)
import jax
import jax.numpy as jnp
from jax.experimental import pallas as pl
from jax.experimental.pallas import tpu as pltpu

SEQ = 512
BAT = 64
HID = 768
T = 32
TT = T * T
START = 30
END = 31
BS = 8
NBLK = SEQ // BS
ROWS = BS * BAT
LOG_T = 3.4657359027997265


def _phase(k, feats_ref, tgt_ref, msk_ref, wt_ref, b_ref, e_ref, s_ref,
           prod_scr, cons_scr, q_ref, o_ref, tg_ref):
    fb = feats_ref[...].astype(jnp.bfloat16)
    prod_scr[...] = (jnp.dot(fb, wt_ref[...], preferred_element_type=jnp.float32)
                     + b_ref[...])

    lane = jax.lax.broadcasted_iota(jnp.int32, (BAT, TT), 1)
    tgt2 = tgt_ref[0]
    msk2 = msk_ref[0]
    tg = jnp.where(k == 1, 0.0, tg_ref[0, 0])
    scb, bounds, mcols = [], [], []
    for i in range(BS):
        sc = cons_scr[i * BAT:(i + 1) * BAT, :]
        tcol = jax.lax.slice(tgt2, (0, i), (BAT, i + 1))
        mcol = jax.lax.slice(msk2, (0, i), (BAT, i + 1))
        tg = tg + jnp.sum(jnp.where((lane == tcol) & (mcol > 0.0), sc, 0.0))
        bound = jnp.max(sc, axis=1, keepdims=True) + LOG_T
        scb.append(sc - bound)
        bounds.append(bound)
        mcols.append(mcol > 0.0)
    tg_ref[0, 0] = tg

    q = q_ref[...]
    o = o_ref[...]
    for i in range(BS):
        pexp = jnp.dot(q, e_ref[...], preferred_element_type=jnp.float32)
        ex = jnp.exp(scb[i] + pexp)
        ssum = jnp.dot(ex, s_ref[...], preferred_element_type=jnp.float32)
        qn = jnp.where(mcols[i], jnp.maximum(jnp.log(ssum), -1e30), q)
        on = jnp.where(mcols[i], o + bounds[i], o)
        if i == 0:
            init_q = jax.lax.slice(scb[0], (0, START * T), (BAT, START * T + T))
            qn = jnp.where(k == 1, init_q, qn)
            on = jnp.where(k == 1, jnp.zeros_like(on) + bounds[0], on)
        q, o = qn, on
    qmax = jnp.max(q, axis=1, keepdims=True)
    q_ref[...] = q - qmax
    o_ref[...] = o + qmax


def _crf_body(feats_ref, tgt_ref, msk_ref, wt_ref, b_ref, e_ref, s_ref,
              out_ref, sc_a, sc_b, q_ref, o_ref, tg_ref):
    k = pl.program_id(0)
    p = jax.lax.rem(k, 2)

    @pl.when(p == 0)
    def _():
        _phase(k, feats_ref, tgt_ref, msk_ref, wt_ref, b_ref, e_ref, s_ref,
               sc_a, sc_b, q_ref, o_ref, tg_ref)

    @pl.when(p == 1)
    def _():
        _phase(k, feats_ref, tgt_ref, msk_ref, wt_ref, b_ref, e_ref, s_ref,
               sc_b, sc_a, q_ref, o_ref, tg_ref)

    @pl.when(k == NBLK)
    def _():
        pend = q_ref[...] + o_ref[...]
        logz = jnp.sum(jax.lax.slice(pend, (0, END), (BAT, END + 1)))
        out_ref[0, 0] = (logz - tg_ref[0, 0]) / float(BAT)


def kernel(feats, target, mask, W, b):
    feats2 = feats.reshape(SEQ * BAT, HID)
    wt = W.T.astype(jnp.bfloat16)
    b2 = b.reshape(1, TT)
    tgt = target[..., 0].astype(jnp.int32).reshape(NBLK, BS, BAT).transpose(0, 2, 1)
    msk = mask.astype(jnp.float32).reshape(NBLK, BS, BAT).transpose(0, 2, 1)
    jj = jnp.arange(TT, dtype=jnp.int32)
    e_mat = (jj[None, :] // T == jnp.arange(T, dtype=jnp.int32)[:, None]).astype(jnp.float32)
    s_mat = (jj[:, None] % T == jnp.arange(T, dtype=jnp.int32)[None, :]).astype(jnp.float32)

    out = pl.pallas_call(
        _crf_body,
        grid=(NBLK + 1,),
        in_specs=[
            pl.BlockSpec((ROWS, HID), lambda k: (jnp.minimum(k, NBLK - 1), 0)),
            pl.BlockSpec((1, BAT, BS), lambda k: (jnp.maximum(k - 1, 0), 0, 0)),
            pl.BlockSpec((1, BAT, BS), lambda k: (jnp.maximum(k - 1, 0), 0, 0)),
            pl.BlockSpec((HID, TT), lambda k: (0, 0)),
            pl.BlockSpec((1, TT), lambda k: (0, 0)),
            pl.BlockSpec((T, TT), lambda k: (0, 0)),
            pl.BlockSpec((TT, T), lambda k: (0, 0)),
        ],
        out_specs=pl.BlockSpec((1, 1), lambda k: (0, 0), memory_space=pltpu.SMEM),
        out_shape=jax.ShapeDtypeStruct((1, 1), jnp.float32),
        scratch_shapes=[
            pltpu.VMEM((ROWS, TT), jnp.float32),
            pltpu.VMEM((ROWS, TT), jnp.float32),
            pltpu.VMEM((BAT, T), jnp.float32),
            pltpu.VMEM((BAT, T), jnp.float32),
            pltpu.SMEM((1, 1), jnp.float32),
        ],
        compiler_params=pltpu.CompilerParams(dimension_semantics=("arbitrary",)),
    )(feats2, tgt, msk, wt, b2, e_mat, s_mat)
    return out[0, 0]


# speedup vs baseline: 1.0356x; 1.0356x over previous
import jax
import jax.numpy as jnp
from jax.experimental import pallas as pl
from jax.experimental.pallas import tpu as pltpu

SEQ = 512
BAT = 64
HID = 768
T = 32
TT = T * T
START = 30
END = 31
BS = 8
NBLK = SEQ // BS
ROWS = BS * BAT
HB = BAT // 2
LOG_T = 3.4657359027997265


def _phase(k, feats_ref, tgt_ref, msk_ref, wt_ref, b_ref, e_ref, s_ref,
           prod_scr, cons_scr, p_ref, o_ref, tg_ref):
    fb = feats_ref[...].astype(jnp.bfloat16)
    prod_scr[...] = (jnp.dot(fb, wt_ref[...], preferred_element_type=jnp.float32)
                     + b_ref[...])

    lane = jax.lax.broadcasted_iota(jnp.int32, (BAT, TT), 1)
    tgt2 = tgt_ref[0]
    msk2 = msk_ref[0]
    tg = jnp.where(k == 1, 0.0, tg_ref[0, 0])
    exs, bounds, mcols = [], [], []
    for i in range(BS):
        sc = cons_scr[i * BAT:(i + 1) * BAT, :]
        tcol = jax.lax.slice(tgt2, (0, i), (BAT, i + 1))
        mcol = jax.lax.slice(msk2, (0, i), (BAT, i + 1))
        tg = tg + jnp.sum(jnp.where((lane == tcol) & (mcol > 0.0), sc, 0.0))
        bound = jnp.max(sc, axis=1, keepdims=True) + LOG_T
        exs.append(jnp.exp(sc - bound))
        bounds.append(bound)
        mcols.append(mcol > 0.0)
    tg_ref[0, 0] = tg

    e = e_ref[...]
    s = s_ref[...]
    ch = []
    for c in range(2):
        lo = c * HB
        ch.append([p_ref[lo:lo + HB, :], o_ref[lo:lo + HB, :], lo])
    for i in range(BS):
        g = [jnp.dot(ch[c][0], e, preferred_element_type=jnp.float32)
             for c in range(2)]
        ex = [g[c] * jax.lax.slice(exs[i], (ch[c][2], 0), (ch[c][2] + HB, TT))
              for c in range(2)]
        pn = [jnp.dot(ex[c], s, preferred_element_type=jnp.float32)
              for c in range(2)]
        for c in range(2):
            lo = ch[c][2]
            mcol = jax.lax.slice(mcols[i], (lo, 0), (lo + HB, 1))
            bnd = jax.lax.slice(bounds[i], (lo, 0), (lo + HB, 1))
            pcur, ocur = ch[c][0], ch[c][1]
            pnew = jnp.where(mcol, pn[c], pcur)
            onew = jnp.where(mcol, ocur + bnd, ocur)
            if i == 0:
                init_p = jax.lax.slice(exs[0], (lo, START * T),
                                       (lo + HB, START * T + T))
                pnew = jnp.where(k == 1, init_p, pnew)
                onew = jnp.where(k == 1, jnp.zeros_like(onew) + bnd, onew)
            ch[c][0], ch[c][1] = pnew, onew
    for c in range(2):
        lo = ch[c][2]
        pmax = jnp.max(ch[c][0], axis=1, keepdims=True)
        pmax = jnp.maximum(pmax, 1e-30)
        p_ref[lo:lo + HB, :] = ch[c][0] / pmax
        o_ref[lo:lo + HB, :] = ch[c][1] + jnp.log(pmax)


def _crf_body(feats_ref, tgt_ref, msk_ref, wt_ref, b_ref, e_ref, s_ref,
              out_ref, sc_a, sc_b, p_ref, o_ref, tg_ref):
    k = pl.program_id(0)
    p = jax.lax.rem(k, 2)

    @pl.when(p == 0)
    def _():
        _phase(k, feats_ref, tgt_ref, msk_ref, wt_ref, b_ref, e_ref, s_ref,
               sc_a, sc_b, p_ref, o_ref, tg_ref)

    @pl.when(p == 1)
    def _():
        _phase(k, feats_ref, tgt_ref, msk_ref, wt_ref, b_ref, e_ref, s_ref,
               sc_b, sc_a, p_ref, o_ref, tg_ref)

    @pl.when(k == NBLK)
    def _():
        pend = jnp.log(jnp.maximum(p_ref[...], 1e-38)) + o_ref[...]
        logz = jnp.sum(jax.lax.slice(pend, (0, END), (BAT, END + 1)))
        out_ref[0, 0] = (logz - tg_ref[0, 0]) / float(BAT)


def kernel(feats, target, mask, W, b):
    feats2 = feats.reshape(SEQ * BAT, HID)
    wt = W.T.astype(jnp.bfloat16)
    b2 = b.reshape(1, TT)
    tgt = target[..., 0].astype(jnp.int32).reshape(NBLK, BS, BAT).transpose(0, 2, 1)
    msk = mask.astype(jnp.float32).reshape(NBLK, BS, BAT).transpose(0, 2, 1)
    jj = jnp.arange(TT, dtype=jnp.int32)
    e_mat = (jj[None, :] // T == jnp.arange(T, dtype=jnp.int32)[:, None]).astype(jnp.float32)
    s_mat = (jj[:, None] % T == jnp.arange(T, dtype=jnp.int32)[None, :]).astype(jnp.float32)

    out = pl.pallas_call(
        _crf_body,
        grid=(NBLK + 1,),
        in_specs=[
            pl.BlockSpec((ROWS, HID), lambda k: (jnp.minimum(k, NBLK - 1), 0)),
            pl.BlockSpec((1, BAT, BS), lambda k: (jnp.maximum(k - 1, 0), 0, 0)),
            pl.BlockSpec((1, BAT, BS), lambda k: (jnp.maximum(k - 1, 0), 0, 0)),
            pl.BlockSpec((HID, TT), lambda k: (0, 0)),
            pl.BlockSpec((1, TT), lambda k: (0, 0)),
            pl.BlockSpec((T, TT), lambda k: (0, 0)),
            pl.BlockSpec((TT, T), lambda k: (0, 0)),
        ],
        out_specs=pl.BlockSpec((1, 1), lambda k: (0, 0), memory_space=pltpu.SMEM),
        out_shape=jax.ShapeDtypeStruct((1, 1), jnp.float32),
        scratch_shapes=[
            pltpu.VMEM((ROWS, TT), jnp.float32),
            pltpu.VMEM((ROWS, TT), jnp.float32),
            pltpu.VMEM((BAT, T), jnp.float32),
            pltpu.VMEM((BAT, T), jnp.float32),
            pltpu.SMEM((1, 1), jnp.float32),
        ],
        compiler_params=pltpu.CompilerParams(dimension_semantics=("arbitrary",)),
    )(feats2, tgt, msk, wt, b2, e_mat, s_mat)
    return out[0, 0]
